# baseline (speedup 1.0000x reference)
import functools

import jax
import jax.numpy as jnp
from jax import lax
from jax.experimental import pallas as pl
from jax.experimental.pallas import tpu as pltpu
from jax.experimental.pallas import tpu_sc as plsc

MOMENTUM = 0.99
N_BINS = 512
L = 16
NC, NS = 2, 16
NW = NC * NS
CH = 192
BATCH = 2
H, W = 256, 256
CPT = CH // NW
HCH = 32
CHUNK = HCH * W
NBLK = H // HCH
NPC = BATCH * NBLK
NELEM = BATCH * H * W
GRP = W // (2 * L)
MAGIC = 1.5 * (2.0 ** 23)
HSTRIDE = N_BINS + 1


def _sc_quant(x, ep_flat):
  mesh = plsc.VectorSubcoreMesh(core_axis_name="c", subcore_axis_name="s")

  @functools.partial(
      pl.kernel,
      out_type=(jax.ShapeDtypeStruct((BATCH, CH, H, W), jnp.float32),
                jax.ShapeDtypeStruct((BATCH, CH, H, W), jnp.float32)),
      mesh=mesh,
      compiler_params=pltpu.CompilerParams(needs_layout_passes=False),
      scratch_types=[
          pltpu.VMEM((HCH, W), jnp.float32),
          pltpu.VMEM((HCH, W), jnp.float32),
          pltpu.VMEM((HCH, W), jnp.float32),
          pltpu.VMEM((HCH, W), jnp.float32),
          pltpu.VMEM((HCH, W), jnp.float32),
          pltpu.VMEM((HCH, W), jnp.float32),
          pltpu.VMEM((NELEM // 2,), jnp.int32),
          pltpu.VMEM((L * HSTRIDE,), jnp.float32),
          pltpu.VMEM((N_BINS,), jnp.float32),
          pltpu.VMEM((CPT * N_BINS,), jnp.float32),
          pltpu.SemaphoreType.DMA,
          pltpu.SemaphoreType.DMA,
          pltpu.SemaphoreType.DMA,
          pltpu.SemaphoreType.DMA,
          pltpu.SemaphoreType.DMA,
          pltpu.SemaphoreType.DMA,
      ],
  )
  def k(x_hbm, ep_hbm, out_hbm, px_hbm,
        bin0, bin1, bout0, bout1, pxb0, pxb1, stash, hist, prow, epall,
        si0, si1, so0, so1, sp0, sp1):
    wid = lax.axis_index("s") * NC + lax.axis_index("c")
    c0 = wid * CPT
    lane_base = lax.iota(jnp.int32, L) * HSTRIDE
    ones = jnp.full((L,), 1.0, jnp.float32)
    zeros16 = jnp.zeros((L,), jnp.float32)
    inv = jnp.float32(1.0 / NELEM)
    one_m = jnp.float32(1.0 - MOMENTUM)
    mom = jnp.float32(MOMENTUM)
    bins = (bin0, bin1)
    bouts = (bout0, bout1)
    pxbs = (pxb0, pxb1)
    sis = (si0, si1)
    sos = (so0, so1)
    sps = (sp0, sp1)

    pltpu.sync_copy(
        ep_hbm.at[pl.ds(pl.multiple_of(c0 * N_BINS, N_BINS), CPT * N_BINS)],
        epall)

    def chunk_at(ref, c, blk):
      b, h0 = blk // NBLK, (blk % NBLK) * HCH
      return ref.at[b, c, pl.ds(h0, HCH), :]

    def zero_hist():
      @plsc.parallel_loop(0, HSTRIDE, unroll=8)
      def _(i):
        hist[pl.ds(i * L, L)] = zeros16

    def p1_body(i, binb, boutb, soff):
      row = lax.shift_right_logical(i, 3)
      off = (i & (GRP - 1)) * 2 * L
      xv0 = binb[row, pl.ds(off, L)]
      xv1 = binb[row, pl.ds(off + L, L)]
      r0 = (jnp.minimum(jnp.maximum(xv0, -255.0), 256.0) + MAGIC) - MAGIC
      r1 = (jnp.minimum(jnp.maximum(xv1, -255.0), 256.0) + MAGIC) - MAGIC
      boutb[row, pl.ds(off, L)] = r0
      boutb[row, pl.ds(off + L, L)] = r1
      idx0 = r0.astype(jnp.int32) + 255
      idx1 = r1.astype(jnp.int32) + 255
      plsc.addupdate_scatter(hist, [lane_base + idx0], ones)
      plsc.addupdate_scatter(hist, [lane_base + idx1], ones)
      stash[pl.ds(soff + i * L, L)] = idx0 | (idx1 << 16)

    def p2_body(i, pxb, soff):
      row = lax.shift_right_logical(i, 3)
      off = (i & (GRP - 1)) * 2 * L
      packed = stash[pl.ds(soff + i * L, L)]
      idx0 = packed & 0xFFFF
      idx1 = lax.shift_right_logical(packed, 16)
      pxb[row, pl.ds(off, L)] = plsc.load_gather(prow, [idx0])
      pxb[row, pl.ds(off + L, L)] = plsc.load_gather(prow, [idx1])

    def ema(kk):
      @plsc.parallel_loop(0, N_BINS // L, unroll=2)
      def _(bb):
        b0 = bb * L
        acc = zeros16
        for lane in range(L):
          acc = acc + hist[pl.ds(lane * HSTRIDE + b0, L)]
        prow[pl.ds(b0, L)] = (epall[pl.ds(kk * N_BINS + b0, L)] * mom
                              + (acc * inv) * one_m)

    zero_hist()
    in_d = [None, None]
    out_d = [None, None]
    in_d[0] = pltpu.async_copy(chunk_at(x_hbm, c0, 0), bins[0], sis[0])
    for blk in range(NPC):
      ib = blk % 2
      if blk + 1 < NPC:
        in_d[1 - ib] = pltpu.async_copy(
            chunk_at(x_hbm, c0, blk + 1), bins[1 - ib], sis[1 - ib])
      in_d[ib].wait()
      if out_d[ib] is not None:
        out_d[ib].wait()
      binb, boutb = bins[ib], bouts[ib]
      soff = blk * (CHUNK // 2)

      @plsc.parallel_loop(0, HCH * GRP, unroll=8)
      def _(i):
        p1_body(i, binb, boutb, soff)

      out_d[ib] = pltpu.async_copy(boutb, chunk_at(out_hbm, c0, blk), sos[ib])
    out_d[0].wait()
    out_d[1].wait()
    ema(0)

    def merged(kk, _):
      c = c0 + kk
      zero_hist()
      in_d = [None, None]
      out_d = [None, None]
      px_d = [None, None]
      in_d[0] = pltpu.async_copy(chunk_at(x_hbm, c, 0), bins[0], sis[0])
      for blk in range(NPC):
        ib = blk % 2
        if blk + 1 < NPC:
          in_d[1 - ib] = pltpu.async_copy(
              chunk_at(x_hbm, c, blk + 1), bins[1 - ib], sis[1 - ib])
        in_d[ib].wait()
        if out_d[ib] is not None:
          out_d[ib].wait()
        if px_d[ib] is not None:
          px_d[ib].wait()
        binb, boutb, pxb = bins[ib], bouts[ib], pxbs[ib]
        soff = blk * (CHUNK // 2)

        @plsc.parallel_loop(0, HCH * GRP, unroll=8)
        def _(i):
          p2_body(i, pxb, soff)
          p1_body(i, binb, boutb, soff)

        out_d[ib] = pltpu.async_copy(boutb, chunk_at(out_hbm, c, blk), sos[ib])
        px_d[ib] = pltpu.async_copy(pxb, chunk_at(px_hbm, c - 1, blk), sps[ib])
      out_d[0].wait()
      out_d[1].wait()
      px_d[0].wait()
      px_d[1].wait()
      ema(kk)
      return 0

    lax.fori_loop(1, CPT, merged, 0)

    px_d = [None, None]
    for blk in range(NPC):
      ib = blk % 2
      if px_d[ib] is not None:
        px_d[ib].wait()
      pxb = pxbs[ib]
      soff = blk * (CHUNK // 2)

      @plsc.parallel_loop(0, HCH * GRP, unroll=8)
      def _(i):
        p2_body(i, pxb, soff)

      px_d[ib] = pltpu.async_copy(
          pxb, chunk_at(px_hbm, c0 + CPT - 1, blk), sps[ib])
    px_d[0].wait()
    px_d[1].wait()

  return k(x, ep_flat)


def kernel(x, estimated_p, dummy):
  return _sc_quant(x, estimated_p.reshape(-1))


# speedup vs baseline: 1.0457x; 1.0457x over previous
import functools

import jax
import jax.numpy as jnp
from jax import lax
from jax.experimental import pallas as pl
from jax.experimental.pallas import tpu as pltpu
from jax.experimental.pallas import tpu_sc as plsc

MOMENTUM = 0.99
N_BINS = 512
L = 16
NC, NS = 2, 16
NW = NC * NS
CH = 192
BATCH = 2
H, W = 256, 256
CPT = CH // NW
HCH = 32
CHUNK = HCH * W
NBLK = H // HCH
NPC = BATCH * NBLK
NELEM = BATCH * H * W
GRP = W // (2 * L)
MAGIC = 1.5 * (2.0 ** 23)
HSTRIDE = N_BINS + 1


def _sc_quant(x, ep_flat):
  mesh = plsc.VectorSubcoreMesh(core_axis_name="c", subcore_axis_name="s")

  @functools.partial(
      pl.kernel,
      out_type=(jax.ShapeDtypeStruct((BATCH, CH, H, W), jnp.float32),
                jax.ShapeDtypeStruct((BATCH, CH, H, W), jnp.float32)),
      mesh=mesh,
      compiler_params=pltpu.CompilerParams(needs_layout_passes=False),
      scratch_types=[
          pltpu.VMEM((HCH, W), jnp.float32),
          pltpu.VMEM((HCH, W), jnp.float32),
          pltpu.VMEM((HCH, W), jnp.float32),
          pltpu.VMEM((HCH, W), jnp.float32),
          pltpu.VMEM((HCH, W), jnp.float32),
          pltpu.VMEM((HCH, W), jnp.float32),
          pltpu.VMEM((NELEM // 2,), jnp.int32),
          pltpu.VMEM((L * HSTRIDE,), jnp.float32),
          pltpu.VMEM((N_BINS,), jnp.float32),
          pltpu.VMEM((N_BINS,), jnp.float32),
          pltpu.VMEM((CPT * N_BINS,), jnp.float32),
          pltpu.SemaphoreType.DMA,
          pltpu.SemaphoreType.DMA,
          pltpu.SemaphoreType.DMA,
          pltpu.SemaphoreType.DMA,
          pltpu.SemaphoreType.DMA,
          pltpu.SemaphoreType.DMA,
      ],
  )
  def k(x_hbm, ep_hbm, out_hbm, px_hbm,
        bin0, bin1, bout0, bout1, pxb0, pxb1, stash, hist, prow, cprev, epall,
        si0, si1, so0, so1, sp0, sp1):
    wid = lax.axis_index("s") * NC + lax.axis_index("c")
    c0 = wid * CPT
    lane_base = lax.iota(jnp.int32, L) * HSTRIDE
    ones = jnp.full((L,), 1.0, jnp.float32)
    zeros16 = jnp.zeros((L,), jnp.float32)
    inv = jnp.float32(1.0 / NELEM)
    one_m = jnp.float32(1.0 - MOMENTUM)
    mom = jnp.float32(MOMENTUM)
    bins = (bin0, bin1)
    bouts = (bout0, bout1)
    pxbs = (pxb0, pxb1)
    sis = (si0, si1)
    sos = (so0, so1)
    sps = (sp0, sp1)

    pltpu.sync_copy(
        ep_hbm.at[pl.ds(pl.multiple_of(c0 * N_BINS, N_BINS), CPT * N_BINS)],
        epall)

    def chunk_at(ref, c, blk):
      b, h0 = blk // NBLK, (blk % NBLK) * HCH
      return ref.at[b, c, pl.ds(h0, HCH), :]

    def zero_hist():
      @plsc.parallel_loop(0, HSTRIDE, unroll=8)
      def _(i):
        hist[pl.ds(i * L, L)] = zeros16

      @plsc.parallel_loop(0, N_BINS // L, unroll=2)
      def _(bb):
        cprev[pl.ds(bb * L, L)] = zeros16

    def p1_body(i, binb, boutb, soff):
      row = lax.shift_right_logical(i, 3)
      off = (i & (GRP - 1)) * 2 * L
      xv0 = binb[row, pl.ds(off, L)]
      xv1 = binb[row, pl.ds(off + L, L)]
      r0 = (jnp.minimum(jnp.maximum(xv0, -255.0), 256.0) + MAGIC) - MAGIC
      r1 = (jnp.minimum(jnp.maximum(xv1, -255.0), 256.0) + MAGIC) - MAGIC
      boutb[row, pl.ds(off, L)] = r0
      boutb[row, pl.ds(off + L, L)] = r1
      idx0 = r0.astype(jnp.int32) + 255
      idx1 = r1.astype(jnp.int32) + 255
      plsc.addupdate_scatter(hist, [lane_base + idx0], ones)
      plsc.addupdate_scatter(hist, [lane_base + idx1], ones)
      stash[pl.ds(soff + i * L, L)] = idx0 | (idx1 << 16)

    def p2_body(i, pxb, soff):
      row = lax.shift_right_logical(i, 3)
      off = (i & (GRP - 1)) * 2 * L
      packed = stash[pl.ds(soff + i * L, L)]
      idx0 = packed & 0xFFFF
      idx1 = lax.shift_right_logical(packed, 16)
      pxb[row, pl.ds(off, L)] = plsc.load_gather(prow, [idx0])
      pxb[row, pl.ds(off + L, L)] = plsc.load_gather(prow, [idx1])

    def ema(kk):
      @plsc.parallel_loop(0, N_BINS // L, unroll=2)
      def _(bb):
        b0 = bb * L
        acc = zeros16
        for lane in range(L):
          acc = acc + hist[pl.ds(lane * HSTRIDE + b0, L)]
        cnt = acc - cprev[pl.ds(b0, L)]
        cprev[pl.ds(b0, L)] = acc
        prow[pl.ds(b0, L)] = (epall[pl.ds(kk * N_BINS + b0, L)] * mom
                              + (cnt * inv) * one_m)

    zero_hist()
    in_d = [None, None]
    out_d = [None, None]
    in_d[0] = pltpu.async_copy(chunk_at(x_hbm, c0, 0), bins[0], sis[0])
    for blk in range(NPC):
      ib = blk % 2
      if blk + 1 < NPC:
        in_d[1 - ib] = pltpu.async_copy(
            chunk_at(x_hbm, c0, blk + 1), bins[1 - ib], sis[1 - ib])
      in_d[ib].wait()
      if out_d[ib] is not None:
        out_d[ib].wait()
      binb, boutb = bins[ib], bouts[ib]
      soff = blk * (CHUNK // 2)

      @plsc.parallel_loop(0, HCH * GRP, unroll=4)
      def _(i):
        p1_body(i, binb, boutb, soff)

      out_d[ib] = pltpu.async_copy(boutb, chunk_at(out_hbm, c0, blk), sos[ib])
    out_d[0].wait()
    out_d[1].wait()
    ema(0)

    def merged(kk, _):
      c = c0 + kk
      in_d = [None, None]
      out_d = [None, None]
      px_d = [None, None]
      in_d[0] = pltpu.async_copy(chunk_at(x_hbm, c, 0), bins[0], sis[0])
      for blk in range(NPC):
        ib = blk % 2
        if blk + 1 < NPC:
          in_d[1 - ib] = pltpu.async_copy(
              chunk_at(x_hbm, c, blk + 1), bins[1 - ib], sis[1 - ib])
        in_d[ib].wait()
        if out_d[ib] is not None:
          out_d[ib].wait()
        if px_d[ib] is not None:
          px_d[ib].wait()
        binb, boutb, pxb = bins[ib], bouts[ib], pxbs[ib]
        soff = blk * (CHUNK // 2)

        @plsc.parallel_loop(0, HCH * GRP, unroll=4)
        def _(i):
          p2_body(i, pxb, soff)
          p1_body(i, binb, boutb, soff)

        out_d[ib] = pltpu.async_copy(boutb, chunk_at(out_hbm, c, blk), sos[ib])
        px_d[ib] = pltpu.async_copy(pxb, chunk_at(px_hbm, c - 1, blk), sps[ib])
      out_d[0].wait()
      out_d[1].wait()
      px_d[0].wait()
      px_d[1].wait()
      ema(kk)
      return 0

    lax.fori_loop(1, CPT, merged, 0)

    px_d = [None, None]
    for blk in range(NPC):
      ib = blk % 2
      if px_d[ib] is not None:
        px_d[ib].wait()
      pxb = pxbs[ib]
      soff = blk * (CHUNK // 2)

      @plsc.parallel_loop(0, HCH * GRP, unroll=4)
      def _(i):
        p2_body(i, pxb, soff)

      px_d[ib] = pltpu.async_copy(
          pxb, chunk_at(px_hbm, c0 + CPT - 1, blk), sps[ib])
    px_d[0].wait()
    px_d[1].wait()

  return k(x, ep_flat)


def kernel(x, estimated_p, dummy):
  return _sc_quant(x, estimated_p.reshape(-1))
